# baseline (speedup 1.0000x reference)
import functools

import jax
import jax.numpy as jnp
from jax.experimental import pallas as pl

NUM_EMBEDDINGS = 8192
EMBEDDING_DIM = 32
COMMITMENT_COST = 0.25
N_TOKENS = 32768
BLOCK_T = 256


def _vq_kernel(z_ref, e_ref, q_ref, loss_ref, idx_ref):
    i = pl.program_id(0)
    z = z_ref[...]
    e = e_ref[...]

    z_norm = jnp.sum(z * z, axis=1, keepdims=True)
    e_norm = jnp.sum(e * e, axis=1)
    mm = jax.lax.dot_general(
        z, e, (((1,), (1,)), ((), ())),
        preferred_element_type=jnp.float32)
    distances = z_norm + e_norm - 2.0 * mm

    min_d = jnp.min(distances, axis=1, keepdims=True)
    col = jax.lax.broadcasted_iota(jnp.int32, distances.shape, 1)
    idx = jnp.min(jnp.where(distances == min_d, col, NUM_EMBEDDINGS),
                  axis=1)
    one_hot = (col == idx[:, None]).astype(jnp.float32)
    q = jax.lax.dot_general(
        one_hot, e, (((1,), (0,)), ((), ())),
        preferred_element_type=jnp.float32)

    diff = q - z
    q_ref[...] = z + diff
    idx_ref[...] = idx

    @pl.when(i == 0)
    def _():
        loss_ref[...] = jnp.zeros_like(loss_ref)

    loss_ref[...] += jnp.sum(diff * diff).reshape(1, 1)

    @pl.when(i == pl.num_programs(0) - 1)
    def _():
        scale = (1.0 + COMMITMENT_COST) / (N_TOKENS * EMBEDDING_DIM)
        loss_ref[...] = loss_ref[...] * scale


@functools.partial(jax.jit, static_argnames=("interpret",))
def kernel(inputs, embedding_weight, interpret=False):
    flat = inputs.reshape(-1, EMBEDDING_DIM)
    grid = (N_TOKENS // BLOCK_T,)
    q, loss, idx = pl.pallas_call(
        _vq_kernel,
        grid=grid,
        in_specs=[
            pl.BlockSpec((BLOCK_T, EMBEDDING_DIM), lambda i: (i, 0)),
            pl.BlockSpec((NUM_EMBEDDINGS, EMBEDDING_DIM), lambda i: (0, 0)),
        ],
        out_specs=[
            pl.BlockSpec((BLOCK_T, EMBEDDING_DIM), lambda i: (i, 0)),
            pl.BlockSpec((1, 1), lambda i: (0, 0)),
            pl.BlockSpec((BLOCK_T,), lambda i: (i,)),
        ],
        out_shape=[
            jax.ShapeDtypeStruct((N_TOKENS, EMBEDDING_DIM), jnp.float32),
            jax.ShapeDtypeStruct((1, 1), jnp.float32),
            jax.ShapeDtypeStruct((N_TOKENS,), jnp.int32),
        ],
        interpret=interpret,
    )(flat, embedding_weight)
    return q, loss[0, 0], idx
